# baseline (speedup 1.0000x reference)
import functools

import jax
import jax.numpy as jnp
from jax import lax
from jax.experimental import pallas as pl
from jax.experimental.pallas import tpu as pltpu
from jax.experimental.pallas import tpu_sc as plsc

N_NODES = 100000
K = 32
D = 128
NEG_SLOPE = 0.2

NC, NS, L = 2, 16, 16
NW = NC * NS
B_PAD = 10240
D_PER_W = B_PAD // NW
CHUNK_D = 4
EDGES_PER_CHUNK = CHUNK_D * K
N_CHUNKS = D_PER_W // CHUNK_D

ROWS_BLK = 20000


def _proj_body(t_ref, w_ref, a_ref, p_ref, s_ref):
    t = t_ref[...]
    w = w_ref[...]
    p = jax.lax.dot_general(t, w, (((1,), (1,)), ((), ())),
                            preferred_element_type=jnp.float32)
    a = a_ref[...]
    s = jax.lax.dot_general(a, p, (((1,), (1,)), ((), ())),
                            preferred_element_type=jnp.float32)
    s = jnp.where(s >= 0, s, NEG_SLOPE * s)
    p_ref[...] = p.astype(jnp.bfloat16).reshape(ROWS_BLK * D)
    s_ref[...] = s.reshape(1, 1, ROWS_BLK)


def _project(table, W, a_src):
    grid = (N_NODES // ROWS_BLK,)
    return pl.pallas_call(
        _proj_body,
        grid=grid,
        in_specs=[
            pl.BlockSpec((ROWS_BLK, D), lambda i: (i, 0)),
            pl.BlockSpec((D, D), lambda i: (0, 0)),
            pl.BlockSpec((1, D), lambda i: (0, 0)),
        ],
        out_specs=[
            pl.BlockSpec((ROWS_BLK * D,), lambda i: (i,)),
            pl.BlockSpec((1, 1, ROWS_BLK), lambda i: (i, 0, 0)),
        ],
        out_shape=[
            jax.ShapeDtypeStruct((N_NODES * D,), jnp.bfloat16),
            jax.ShapeDtypeStruct((N_NODES // ROWS_BLK, 1, ROWS_BLK),
                                 jnp.float32),
        ],
    )(table, W, a_src.reshape(1, D))


def _sc_body(idx_hbm, fused_hbm, scores_hbm, out_hbm,
             idx_v, rows_v, sc_v, o_v, sem_r, sem_s):
    wid = lax.axis_index("s") * NC + lax.axis_index("c")
    pltpu.sync_copy(idx_hbm.at[wid], idx_v)

    def start_gather(c, buf):
        pltpu.async_copy(
            fused_hbm.at[idx_v.at[c]],
            rows_v.at[pl.ds(buf * EDGES_PER_CHUNK, EDGES_PER_CHUNK)],
            sem_r,
        )
        pltpu.async_copy(scores_hbm.at[idx_v.at[c]], sc_v.at[buf], sem_s)

    def wait_gather(c, buf):
        pltpu.make_async_copy(
            fused_hbm.at[idx_v.at[c]],
            rows_v.at[pl.ds(buf * EDGES_PER_CHUNK, EDGES_PER_CHUNK)],
            sem_r,
        ).wait()
        pltpu.make_async_copy(
            scores_hbm.at[idx_v.at[c]], sc_v.at[buf], sem_s,
        ).wait()

    start_gather(0, 0)

    def chunk_body(c, carry):
        buf = lax.rem(c, 2)

        @pl.when(c + 1 < N_CHUNKS)
        def _():
            start_gather(c + 1, 1 - buf)

        wait_gather(c, buf)

        base_row = buf * EDGES_PER_CHUNK
        lanes = lax.iota(jnp.int32, L)
        for d in range(CHUNK_D):
            s0 = sc_v[buf, pl.ds(d * K, L)]
            s1 = sc_v[buf, pl.ds(d * K + L, L)]
            m = jnp.max(jnp.maximum(s0, s1))
            e0 = jnp.exp(s0 - m)
            e1 = jnp.exp(s1 - m)
            total = jnp.sum(e0) + jnp.sum(e1) + 1e-13
            acce = [jnp.zeros((L,), jnp.float32) for _ in range(D // (2 * L))]
            acco = [jnp.zeros((L,), jnp.float32) for _ in range(D // (2 * L))]
            for k in range(K):
                wk = (e0 if k < L else e1)[k % L]
                row = base_row + d * K + k
                for j in range(D // (2 * L)):
                    v = rows_v[row, pl.ds(j * 2 * L, 2 * L)]
                    ve, vo = plsc.unpack(v, format=plsc.PackFormat.INTERLEAVED)
                    acce[j] = acce[j] + ve * wk
                    acco[j] = acco[j] + vo * wk
            for j in range(D // (2 * L)):
                cols = j * 2 * L + 2 * lanes
                rows_d = jnp.full((L,), d, dtype=jnp.int32)
                plsc.store_scatter(o_v, [rows_d, cols], acce[j] / total)
                plsc.store_scatter(o_v, [rows_d, cols + 1], acco[j] / total)

        pltpu.sync_copy(
            o_v, out_hbm.at[pl.ds(wid * D_PER_W + c * CHUNK_D, CHUNK_D)])
        return carry

    lax.fori_loop(0, N_CHUNKS, chunk_body, 0)


@functools.partial(
    pl.kernel,
    mesh=plsc.VectorSubcoreMesh(core_axis_name="c", subcore_axis_name="s"),
    out_type=jax.ShapeDtypeStruct((B_PAD, D), jnp.float32),
    compiler_params=pltpu.CompilerParams(
        use_tc_tiling_on_sc=False, needs_layout_passes=False),
    scratch_types=[
        pltpu.VMEM((N_CHUNKS, EDGES_PER_CHUNK), jnp.int32),
        pltpu.VMEM((2 * EDGES_PER_CHUNK, D), jnp.bfloat16),
        pltpu.VMEM((2, EDGES_PER_CHUNK), jnp.float32),
        pltpu.VMEM((CHUNK_D, D), jnp.float32),
        pltpu.SemaphoreType.DMA,
        pltpu.SemaphoreType.DMA,
    ],
)
def _sc_aggregate(idx_hbm, fused_hbm, scores_hbm, out_hbm,
                  idx_v, rows_v, sc_v, o_v, sem_r, sem_s):
    _sc_body(idx_hbm, fused_hbm, scores_hbm, out_hbm,
             idx_v, rows_v, sc_v, o_v, sem_r, sem_s)


def kernel(nodes, neigh_idx, mask, table, W, a_src, a_dst):
    del nodes, mask, a_dst
    fused, scores = _project(table, W, a_src)
    fused = fused.reshape(N_NODES, D)
    scores = scores.reshape(N_NODES)
    B = neigh_idx.shape[0]
    pad = (jnp.arange((B_PAD - B) * K, dtype=jnp.int32) * 37) % N_NODES
    idx = jnp.concatenate([neigh_idx, pad.reshape(B_PAD - B, K)])
    idx = idx.reshape(NW, N_CHUNKS, EDGES_PER_CHUNK)
    out = _sc_aggregate(idx, fused, scores)
    return out[:B]


# speedup vs baseline: 1.5320x; 1.5320x over previous
import functools

import jax
import jax.numpy as jnp
from jax import lax
from jax.experimental import pallas as pl
from jax.experimental.pallas import tpu as pltpu
from jax.experimental.pallas import tpu_sc as plsc

N_NODES = 100000
B = 10000
K = 32
D = 128
NEG_SLOPE = 0.2

NC, NS, L = 2, 16, 16
NW = NC * NS
CHUNK_D = 4
EDGES_PER_CHUNK = CHUNK_D * K
N_CHUNKS = B // CHUNK_D
CHUNKS_LO = N_CHUNKS // NW
N_EXTRA = N_CHUNKS - CHUNKS_LO * NW

ROWS_BLK = 20000


def _proj_body(t_ref, w_ref, a_ref, p_ref, s_ref):
    t = t_ref[...]
    w = w_ref[...]
    p = jax.lax.dot_general(t, w, (((1,), (1,)), ((), ())),
                            preferred_element_type=jnp.float32)
    a = a_ref[...]
    s = jax.lax.dot_general(a, p, (((1,), (1,)), ((), ())),
                            preferred_element_type=jnp.float32)
    s = jnp.where(s >= 0, s, NEG_SLOPE * s)
    p_ref[...] = p
    s_ref[...] = s.reshape(1, 1, ROWS_BLK)


def _project(table, W, a_src):
    grid = (N_NODES // ROWS_BLK,)
    return pl.pallas_call(
        _proj_body,
        grid=grid,
        in_specs=[
            pl.BlockSpec((ROWS_BLK, D), lambda i: (i, 0)),
            pl.BlockSpec((D, D), lambda i: (0, 0)),
            pl.BlockSpec((1, D), lambda i: (0, 0)),
        ],
        out_specs=[
            pl.BlockSpec((ROWS_BLK, D), lambda i: (i, 0)),
            pl.BlockSpec((1, 1, ROWS_BLK), lambda i: (i, 0, 0)),
        ],
        out_shape=[
            jax.ShapeDtypeStruct((N_NODES, D), jnp.float32),
            jax.ShapeDtypeStruct((N_NODES // ROWS_BLK, 1, ROWS_BLK),
                                 jnp.float32),
        ],
    )(table, W, a_src.reshape(1, D))


def _sc_body(idx_hbm, fused_hbm, scores_hbm, out_hbm,
             idx_v, rows_v, sc_v, o_v, shared_s, sem_r, sem_s):
    cid = lax.axis_index("c")
    sid = lax.axis_index("s")
    wid = sid * NC + cid
    offs = CHUNKS_LO * wid + jnp.minimum(wid, N_EXTRA)
    nc = jnp.where(wid < N_EXTRA, CHUNKS_LO + 1, CHUNKS_LO)

    @pl.when(wid < N_EXTRA)
    def _():
        pltpu.sync_copy(idx_hbm.at[pl.ds(offs, CHUNKS_LO + 1)], idx_v)

    @pl.when(wid >= N_EXTRA)
    def _():
        pltpu.sync_copy(idx_hbm.at[pl.ds(offs, CHUNKS_LO)],
                        idx_v.at[pl.ds(0, CHUNKS_LO)])

    @pl.when(sid == 0)
    def _():
        pltpu.sync_copy(scores_hbm, shared_s)

    plsc.subcore_barrier()

    def start_gather(c, buf):
        pltpu.async_copy(
            fused_hbm.at[idx_v.at[c]],
            rows_v.at[pl.ds(buf * EDGES_PER_CHUNK, EDGES_PER_CHUNK)],
            sem_r,
        )
        pltpu.async_copy(shared_s.at[idx_v.at[c]], sc_v.at[buf], sem_s)

    def wait_gather(c, buf):
        pltpu.make_async_copy(
            fused_hbm.at[idx_v.at[c]],
            rows_v.at[pl.ds(buf * EDGES_PER_CHUNK, EDGES_PER_CHUNK)],
            sem_r,
        ).wait()
        pltpu.make_async_copy(
            shared_s.at[idx_v.at[c]], sc_v.at[buf], sem_s,
        ).wait()

    start_gather(0, 0)

    def chunk_body(c, carry):
        buf = lax.rem(c, 2)

        @pl.when(c + 1 < nc)
        def _():
            start_gather(c + 1, 1 - buf)

        wait_gather(c, buf)

        base_row = buf * EDGES_PER_CHUNK
        for d in range(CHUNK_D):
            s0 = sc_v[buf, pl.ds(d * K, L)]
            s1 = sc_v[buf, pl.ds(d * K + L, L)]
            m = jnp.max(jnp.maximum(s0, s1))
            e0 = jnp.exp(s0 - m)
            e1 = jnp.exp(s1 - m)
            total = jnp.sum(e0) + jnp.sum(e1) + 1e-13
            accs = [jnp.zeros((L,), jnp.float32) for _ in range(D // L)]
            for k in range(K):
                wk = (e0 if k < L else e1)[k % L]
                row = base_row + d * K + k
                for j in range(D // L):
                    accs[j] = accs[j] + rows_v[row, pl.ds(j * L, L)] * wk
            for j in range(D // L):
                o_v[d, pl.ds(j * L, L)] = accs[j] / total

        pltpu.sync_copy(
            o_v, out_hbm.at[pl.ds((offs + c) * CHUNK_D, CHUNK_D)])
        return carry

    lax.fori_loop(0, nc, chunk_body, 0)


@functools.partial(
    pl.kernel,
    mesh=plsc.VectorSubcoreMesh(core_axis_name="c", subcore_axis_name="s"),
    out_type=jax.ShapeDtypeStruct((B, D), jnp.float32),
    compiler_params=pltpu.CompilerParams(
        use_tc_tiling_on_sc=False, needs_layout_passes=False),
    scratch_types=[
        pltpu.VMEM((CHUNKS_LO + 1, EDGES_PER_CHUNK), jnp.int32),
        pltpu.VMEM((2 * EDGES_PER_CHUNK, D), jnp.float32),
        pltpu.VMEM((2, EDGES_PER_CHUNK), jnp.float32),
        pltpu.VMEM((CHUNK_D, D), jnp.float32),
        pltpu.VMEM_SHARED((N_NODES,), jnp.float32),
        pltpu.SemaphoreType.DMA,
        pltpu.SemaphoreType.DMA,
    ],
)
def _sc_aggregate(idx_hbm, fused_hbm, scores_hbm, out_hbm,
                  idx_v, rows_v, sc_v, o_v, shared_s, sem_r, sem_s):
    _sc_body(idx_hbm, fused_hbm, scores_hbm, out_hbm,
             idx_v, rows_v, sc_v, o_v, shared_s, sem_r, sem_s)


def kernel(nodes, neigh_idx, mask, table, W, a_src, a_dst):
    del nodes, mask, a_dst
    fused, scores = _project(table, W, a_src)
    scores = scores.reshape(N_NODES)
    idx = neigh_idx.reshape(N_CHUNKS, EDGES_PER_CHUNK)
    return _sc_aggregate(idx, fused, scores)
